# Initial kernel scaffold; baseline (speedup 1.0000x reference)
import jax
import jax.numpy as jnp
from jax.experimental import pallas as pl


def kernel(input_ids, table, ti_emb, offsets):
    raise NotImplementedError("write your pallas kernel here")


# speedup vs baseline: 8.0436x; 8.0436x over previous
import functools

import jax
import jax.numpy as jnp
from jax import lax
from jax.experimental import pallas as pl
from jax.experimental.pallas import tpu as pltpu
from jax.experimental.pallas import tpu_sc as plsc

VOCAB = 49408
B = 256
L = 77
D = 1024
NVEC = 8

NC = 2
NS = 16
NW = NC * NS
N = B * L
PER_W = N // NW
BPW = B // NW
CHUNK = 48
NFULL = PER_W // CHUNK
TAIL = PER_W - NFULL * CHUNK


def _sc_kernel(ids_hbm, table_hbm, ti2_hbm, dest_hbm, out_hbm,
               ids_v, buf0, buf1, ti_v, didx_v, gsem, dsem):
    wid = lax.axis_index("s") * NC + lax.axis_index("c")
    base = wid * PER_W

    pltpu.sync_copy(ids_hbm.at[pl.ds(base, PER_W)], ids_v)
    pltpu.sync_copy(ti2_hbm, ti_v)

    bufs = (buf0, buf1)
    nchunks = NFULL + 1

    def gather(c, size):
        return pltpu.async_copy(
            table_hbm.at[ids_v.at[pl.ds(c * CHUNK, size)]],
            bufs[c % 2].at[pl.ds(0, size)], gsem)

    def writeback(c, size):
        pltpu.sync_copy(bufs[c % 2].at[pl.ds(0, size)],
                        out_hbm.at[pl.ds(base + c * CHUNK, size)])

    sizes = [CHUNK] * NFULL + [TAIL]
    copies = [gather(0, sizes[0])]
    for c in range(1, nchunks):
        copies.append(gather(c, sizes[c]))
        copies[c - 1].wait()
        writeback(c - 1, sizes[c - 1])
    copies[nchunks - 1].wait()
    writeback(nchunks - 1, sizes[nchunks - 1])

    for g in range(BPW // 2):
        pltpu.sync_copy(dest_hbm.at[wid, g], didx_v)
        pltpu.async_copy(ti_v, out_hbm.at[didx_v], dsem).wait()


@jax.jit
def kernel(input_ids, table, ti_emb, offsets):
    ids_flat = input_ids.reshape(N)
    ti2 = jnp.concatenate([ti_emb, ti_emb], axis=0)
    lane = jnp.arange(16, dtype=jnp.int32)
    g = jnp.arange(BPW // 2, dtype=jnp.int32)
    lb = g[None, :, None] * 2 + (lane[None, None, :] >> 3)
    prompt = jnp.arange(NW, dtype=jnp.int32)[:, None, None] * BPW + lb
    off = offsets[prompt]
    dest = prompt * L + off + 1 + (lane[None, None, :] & 7)

    mesh = plsc.VectorSubcoreMesh(core_axis_name="c", subcore_axis_name="s")
    out = pl.kernel(
        _sc_kernel,
        out_type=jax.ShapeDtypeStruct((N, D), jnp.float32),
        mesh=mesh,
        scratch_types=[
            pltpu.VMEM((PER_W,), jnp.int32),
            pltpu.VMEM((CHUNK, D), jnp.float32),
            pltpu.VMEM((CHUNK, D), jnp.float32),
            pltpu.VMEM((16, D), jnp.float32),
            pltpu.VMEM((16,), jnp.int32),
            pltpu.SemaphoreType.DMA,
            pltpu.SemaphoreType.DMA,
        ],
    )(ids_flat, table, ti2, dest)
    return out.reshape(B, L, D)
